# baseline (speedup 1.0000x reference)
import functools
import math

import jax
import jax.numpy as jnp
import numpy as np
from jax import lax
from jax.experimental import pallas as pl
from jax.experimental.pallas import tpu as pltpu
from jax.experimental.pallas import tpu_sc as plsc

VOCAB = 1000000
D = 64
L = 200
B = 4096
N = B * L

NC = 2
NS = 16
NW = NC * NS
S = N // NW
C = 128
NCH = S // C
NBUF = 4
SCALE = math.sqrt(float(D))


def _sinusoidal_pe_np(max_len, d):
    pos = np.arange(max_len, dtype=np.float32)[:, None]
    div = np.exp(np.arange(0, d, 2, dtype=np.float32) * (-np.log(10000.0) / d))
    pe = np.zeros((max_len, d), dtype=np.float32)
    pe[:, 0::2] = np.sin(pos * div)
    pe[:, 1::2] = np.cos(pos * div)
    return pe


_PE2 = np.tile(_sinusoidal_pe_np(L, D), (2, 1)).reshape(L, 2 * D)


def _sc_body(table_hbm, idx_hbm, pe_hbm, out_hbm,
             idx_v, pe_v, in_v, out_v,
             g0, g1, g2, g3, s0, s1, s2, s3):
    gsem = [g0, g1, g2, g3]
    ssem = [s0, s1, s2, s3]
    wid = lax.axis_index("s") * NC + lax.axis_index("c")
    base = wid * S

    pltpu.sync_copy(idx_hbm.at[wid], idx_v)
    pltpu.sync_copy(pe_hbm, pe_v)

    for j in range(NBUF):
        pltpu.make_async_copy(
            table_hbm.at[idx_v.at[j]], in_v.at[j], gsem[j]).start()

    def outer(o, carry):
        for j in range(NBUF):
            k = o * NBUF + j
            pltpu.make_async_copy(
                table_hbm.at[idx_v.at[k]], in_v.at[j], gsem[j]).wait()
            @pl.when(k >= NBUF)
            def _():
                pltpu.make_async_copy(
                    out_v.at[j], out_hbm.at[pl.ds(base // 2, C // 2)],
                    ssem[j]).wait()

            pq = lax.rem(k * C, L) // 2

            def fma_pairs(q0, carry2):
                for u in range(4):
                    q = q0 + u
                    r0 = 2 * q
                    for s in range(8):
                        row = in_v[j, r0 + s // 4, pl.ds((s % 4) * 16, 16)]
                        pev = pe_v[pq + q, pl.ds(s * 16, 16)]
                        out_v[j, q, pl.ds(s * 16, 16)] = row * SCALE + pev
                return carry2

            lax.fori_loop(0, C // 8, lambda q, c2: fma_pairs(q * 4, c2), 0,
                          unroll=False)

            @pl.when(k + NBUF < NCH)
            def _():
                pltpu.make_async_copy(
                    table_hbm.at[idx_v.at[k + NBUF]], in_v.at[j],
                    gsem[j]).start()

            pltpu.make_async_copy(
                out_v.at[j], out_hbm.at[pl.ds((base + k * C) // 2, C // 2)],
                ssem[j]).start()
        return carry

    lax.fori_loop(0, NCH // NBUF, outer, 0, unroll=False)

    for j in range(NBUF):
        pltpu.make_async_copy(
            out_v.at[j], out_hbm.at[pl.ds(base // 2, C // 2)], ssem[j]).wait()


def kernel(tokens, table):
    idx = tokens.astype(jnp.int32).reshape(NW, NCH, C)
    pe2 = jnp.asarray(_PE2)

    mesh = plsc.VectorSubcoreMesh(core_axis_name="c", subcore_axis_name="s")
    run = functools.partial(
        pl.kernel,
        mesh=mesh,
        compiler_params=pltpu.CompilerParams(use_tc_tiling_on_sc=False),
        out_type=jax.ShapeDtypeStruct((N // 2, 2 * D), jnp.float32),
        scratch_types=[
            pltpu.VMEM((NCH, C), jnp.int32),
            pltpu.VMEM((L, 2 * D), jnp.float32),
            pltpu.VMEM((NBUF, C, D), jnp.float32),
            pltpu.VMEM((NBUF, C // 2, 2 * D), jnp.float32),
        ] + [pltpu.SemaphoreType.DMA] * (2 * NBUF),
    )(_sc_body)

    out = run(table, idx, pe2)
    return out.reshape(B, L, D)


# speedup vs baseline: 1.3158x; 1.3158x over previous
import functools
import math

import jax
import jax.numpy as jnp
import numpy as np
from jax import lax
from jax.experimental import pallas as pl
from jax.experimental.pallas import tpu as pltpu
from jax.experimental.pallas import tpu_sc as plsc

VOCAB = 1000000
D = 64
L = 200
B = 4096
N = B * L

NC = 2
NS = 16
NW = NC * NS
S = N // NW
C = 128
NCH = S // C
NBUF = 4
SCALE = math.sqrt(float(D))


def _sinusoidal_pe_np(max_len, d):
    pos = np.arange(max_len, dtype=np.float32)[:, None]
    div = np.exp(np.arange(0, d, 2, dtype=np.float32) * (-np.log(10000.0) / d))
    pe = np.zeros((max_len, d), dtype=np.float32)
    pe[:, 0::2] = np.sin(pos * div)
    pe[:, 1::2] = np.cos(pos * div)
    return pe


_PE2 = np.tile(_sinusoidal_pe_np(L, D), (2, 1)).reshape(L, 2 * D)


def _sc_body(table_hbm, idx_hbm, pe_hbm, out_hbm,
             idx_v, pe_v, in_v, out_v,
             g0, g1, g2, g3, s0, s1, s2, s3):
    gsem = [g0, g1, g2, g3]
    ssem = [s0, s1, s2, s3]
    wid = lax.axis_index("s") * NC + lax.axis_index("c")
    base = wid * S

    pltpu.sync_copy(idx_hbm.at[wid], idx_v)
    pltpu.sync_copy(pe_hbm, pe_v)

    for j in range(NBUF):
        pltpu.make_async_copy(
            table_hbm.at[idx_v.at[j]], in_v.at[j], gsem[j]).start()

    def outer(o, carry):
        for j in range(NBUF):
            k = o * NBUF + j
            pltpu.make_async_copy(
                table_hbm.at[idx_v.at[k]], in_v.at[j], gsem[j]).wait()
            @pl.when(k >= NBUF)
            def _():
                pltpu.make_async_copy(
                    out_v.at[j], out_hbm.at[pl.ds(base // 2, C // 2)],
                    ssem[j]).wait()

            pq = lax.rem(k * C, L) // 2

            @plsc.parallel_loop(0, C // 2, step=1, unroll=4)
            def fma_pairs(q):
                r0 = 2 * q
                for s in range(8):
                    row = in_v[j, r0 + s // 4, pl.ds((s % 4) * 16, 16)]
                    pev = pe_v[pq + q, pl.ds(s * 16, 16)]
                    out_v[j, q, pl.ds(s * 16, 16)] = row * SCALE + pev

            @pl.when(k + NBUF < NCH)
            def _():
                pltpu.make_async_copy(
                    table_hbm.at[idx_v.at[k + NBUF]], in_v.at[j],
                    gsem[j]).start()

            pltpu.make_async_copy(
                out_v.at[j], out_hbm.at[pl.ds((base + k * C) // 2, C // 2)],
                ssem[j]).start()
        return carry

    lax.fori_loop(0, NCH // NBUF, outer, 0, unroll=False)

    for j in range(NBUF):
        pltpu.make_async_copy(
            out_v.at[j], out_hbm.at[pl.ds(base // 2, C // 2)], ssem[j]).wait()


def kernel(tokens, table):
    idx = tokens.astype(jnp.int32).reshape(NW, NCH, C)
    pe2 = jnp.asarray(_PE2)

    mesh = plsc.VectorSubcoreMesh(core_axis_name="c", subcore_axis_name="s")
    run = functools.partial(
        pl.kernel,
        mesh=mesh,
        compiler_params=pltpu.CompilerParams(use_tc_tiling_on_sc=False),
        out_type=jax.ShapeDtypeStruct((N // 2, 2 * D), jnp.float32),
        scratch_types=[
            pltpu.VMEM((NCH, C), jnp.int32),
            pltpu.VMEM((L, 2 * D), jnp.float32),
            pltpu.VMEM((NBUF, C, D), jnp.float32),
            pltpu.VMEM((NBUF, C // 2, 2 * D), jnp.float32),
        ] + [pltpu.SemaphoreType.DMA] * (2 * NBUF),
    )(_sc_body)

    out = run(table, idx, pe2)
    return out.reshape(B, L, D)
